# baseline (speedup 1.0000x reference)
import functools

import jax
import jax.numpy as jnp
from jax import lax
from jax.experimental import pallas as pl
from jax.experimental.pallas import tpu as pltpu
from jax.experimental.pallas import tpu_sc as plsc

B = 1024
DENSE_DIM = 13
SPARSE_DIM = 26
VOCAB = 1000
EMB = 64
FN = DENSE_DIM + SPARSE_DIM * EMB

NC, NS = 2, 16
NW = NC * NS
B_PER_W = B // NW

_sc_mesh = plsc.VectorSubcoreMesh(
    core_axis_name="c", subcore_axis_name="s", num_cores=NC, num_subcores=NS)


@functools.partial(
    pl.kernel,
    mesh=_sc_mesh,
    out_type=jax.ShapeDtypeStruct((SPARSE_DIM, B, 128), jnp.float32),
    scratch_types=[
        pltpu.VMEM((SPARSE_DIM, 128), jnp.int32),
        pltpu.VMEM((SPARSE_DIM, B_PER_W, 128), jnp.float32),
        pltpu.SemaphoreType.DMA,
        pltpu.SemaphoreType.DMA,
    ],
    compiler_params=pltpu.CompilerParams(use_tc_tiling_on_sc=True),
)
def _sc_gather(table_hbm, idx_hbm, out_hbm, idx_v, rows_v, gsem, wsem):
    wid = lax.axis_index("s") * NC + lax.axis_index("c")
    pltpu.sync_copy(idx_hbm.at[:, wid // 4], idx_v)
    lane0 = (wid % 4) * B_PER_W
    gathers = []
    for f in range(SPARSE_DIM):
        gathers.append(
            pltpu.async_copy(
                table_hbm.at[f].at[idx_v.at[f, pl.ds(lane0, B_PER_W)]],
                rows_v.at[f], gsem))
    for g in gathers:
        g.wait()
    writes = []
    for f in range(SPARSE_DIM):
        writes.append(
            pltpu.async_copy(rows_v.at[f],
                             out_hbm.at[f].at[pl.ds(wid * B_PER_W, B_PER_W)],
                             wsem))
    for wr in writes:
        wr.wait()


def _tp_body(tt_ref, out_ref):
    out_ref[0, :, :EMB] = jnp.transpose(tt_ref[0])


_tp_call = pl.pallas_call(
    _tp_body,
    grid=(SPARSE_DIM,),
    in_specs=[pl.BlockSpec((1, EMB, VOCAB), lambda i: (i, 0, 0))],
    out_specs=pl.BlockSpec((1, VOCAB, 128), lambda i: (i, 0, 0)),
    out_shape=jax.ShapeDtypeStruct((SPARSE_DIM, VOCAB, 128), jnp.float32),
    compiler_params=pltpu.CompilerParams(
        dimension_semantics=("parallel",)),
)


def _tc_body(emb_ref, inp_ref, w0_ref, w_ref, V_ref, W1b, b1_ref, W2b,
             b2_ref, W3b, b3_ref, Wo_ref, bo_ref, out_ref):
    parts = [inp_ref[:, :DENSE_DIM]]
    for f in range(SPARSE_DIM):
        parts.append(emb_ref[f, :, :EMB])
    x = jnp.concatenate(parts, axis=1)
    V = V_ref[...]
    xv = jnp.dot(x, V, preferred_element_type=jnp.float32)
    s1 = jnp.sum(xv * xv, axis=1, keepdims=True)
    v2s = jnp.sum(V * V, axis=1, keepdims=True)
    s2 = jnp.dot(x * x, v2s, preferred_element_type=jnp.float32)
    lin = jnp.dot(x, w_ref[...], preferred_element_type=jnp.float32)
    fm = w0_ref[0, 0] + lin + 0.5 * (s1 - s2)
    xb = x.astype(jnp.bfloat16)
    h = jnp.maximum(
        jnp.dot(xb, W1b[...], preferred_element_type=jnp.float32)
        + b1_ref[...], 0.0)
    h = jnp.maximum(
        jnp.dot(h.astype(jnp.bfloat16), W2b[...],
                preferred_element_type=jnp.float32) + b2_ref[...], 0.0)
    h = jnp.maximum(
        jnp.dot(h.astype(jnp.bfloat16), W3b[...],
                preferred_element_type=jnp.float32) + b3_ref[...], 0.0)
    deep = jnp.dot(h, Wo_ref[...], preferred_element_type=jnp.float32) + bo_ref[0, 0]
    out_ref[...] = jax.nn.sigmoid(0.5 * (fm + deep))


_BLK = 1024
_H1, _H2, _H3 = 1024, 512, 256


def _full(shape):
    n = len(shape)
    return pl.BlockSpec(shape, lambda i, n=n: (0,) * n)


_tc_call = pl.pallas_call(
    _tc_body,
    grid=(B // _BLK,),
    in_specs=[
        pl.BlockSpec((SPARSE_DIM, _BLK, 128), lambda i: (0, i, 0)),
        pl.BlockSpec((_BLK, DENSE_DIM + SPARSE_DIM), lambda i: (i, 0)),
        _full((1, 1)),
        _full((FN, 1)),
        _full((FN, 64)),
        _full((FN, _H1)),
        _full((1, _H1)),
        _full((_H1, _H2)),
        _full((1, _H2)),
        _full((_H2, _H3)),
        _full((1, _H3)),
        _full((_H3, 1)),
        _full((1, 1)),
    ],
    out_specs=pl.BlockSpec((_BLK, 1), lambda i: (i, 0)),
    out_shape=jax.ShapeDtypeStruct((B, 1), jnp.float32),
    compiler_params=pltpu.CompilerParams(
        dimension_semantics=("arbitrary",)),
)


def kernel(inputs, tables, w0, w, V, W1, b1, W2, b2, W3, b3, Wo, bo):
    table_pad = jnp.pad(tables, ((0, 0), (0, 0), (0, 128 - EMB)))
    idx3 = (inputs[:, DENSE_DIM:].astype(jnp.int32).T
            .reshape(SPARSE_DIM, 8, 128))
    emb3 = _sc_gather(table_pad, idx3)
    out = _tc_call(emb3, inputs, w0.reshape(1, 1), w, V,
                   W1.astype(jnp.bfloat16), b1.reshape(1, _H1),
                   W2.astype(jnp.bfloat16), b2.reshape(1, _H2),
                   W3.astype(jnp.bfloat16), b3.reshape(1, _H3),
                   Wo, bo.reshape(1, 1))
    return out


# speedup vs baseline: 1.0205x; 1.0205x over previous
import functools

import jax
import jax.numpy as jnp
from jax import lax
from jax.experimental import pallas as pl
from jax.experimental.pallas import tpu as pltpu
from jax.experimental.pallas import tpu_sc as plsc

B = 1024
DENSE_DIM = 13
SPARSE_DIM = 26
VOCAB = 1000
EMB = 64
FN = DENSE_DIM + SPARSE_DIM * EMB

NC, NS = 2, 16
NW = NC * NS
B_PER_W = B // NW

_sc_mesh = plsc.VectorSubcoreMesh(
    core_axis_name="c", subcore_axis_name="s", num_cores=NC, num_subcores=NS)


@functools.partial(
    pl.kernel,
    mesh=_sc_mesh,
    out_type=jax.ShapeDtypeStruct((SPARSE_DIM, B, 128), jnp.float32),
    scratch_types=[
        pltpu.VMEM((SPARSE_DIM, 128), jnp.int32),
        pltpu.VMEM((SPARSE_DIM, B_PER_W, 128), jnp.float32),
        pltpu.SemaphoreType.DMA,
        pltpu.SemaphoreType.DMA,
    ],
    compiler_params=pltpu.CompilerParams(use_tc_tiling_on_sc=True),
)
def _sc_gather(table_hbm, idx_hbm, out_hbm, idx_v, rows_v, gsem, wsem):
    wid = lax.axis_index("s") * NC + lax.axis_index("c")
    pltpu.sync_copy(idx_hbm.at[:, wid // 4], idx_v)
    lane0 = (wid % 4) * B_PER_W
    gathers = []
    for f in range(SPARSE_DIM):
        gathers.append(
            pltpu.async_copy(
                table_hbm.at[f].at[idx_v.at[f, pl.ds(lane0, B_PER_W)]],
                rows_v.at[f], gsem))
    for g in gathers:
        g.wait()
    writes = []
    for f in range(SPARSE_DIM):
        writes.append(
            pltpu.async_copy(rows_v.at[f],
                             out_hbm.at[f].at[pl.ds(wid * B_PER_W, B_PER_W)],
                             wsem))
    for wr in writes:
        wr.wait()


def _tc_body(emb_ref, inp_ref, w0_ref, w_ref, V_ref, W1_ref, b1_ref, W2_ref,
             b2_ref, W3_ref, b3_ref, Wo_ref, bo_ref, out_ref):
    parts = [inp_ref[:, :DENSE_DIM]]
    for f in range(SPARSE_DIM):
        parts.append(emb_ref[f, :, :EMB])
    x = jnp.concatenate(parts, axis=1)
    V = V_ref[...]
    xv = jnp.dot(x, V, preferred_element_type=jnp.float32)
    s1 = jnp.sum(xv * xv, axis=1, keepdims=True)
    v2s = jnp.sum(V * V, axis=1, keepdims=True)
    s2 = jnp.dot(x * x, v2s, preferred_element_type=jnp.float32)
    lin = jnp.dot(x, w_ref[...], preferred_element_type=jnp.float32)
    fm = w0_ref[0, 0] + lin + 0.5 * (s1 - s2)
    xb = x.astype(jnp.bfloat16)
    h = jnp.maximum(
        jnp.dot(xb, W1_ref[...].astype(jnp.bfloat16),
                preferred_element_type=jnp.float32) + b1_ref[...], 0.0)
    h = jnp.maximum(
        jnp.dot(h.astype(jnp.bfloat16), W2_ref[...].astype(jnp.bfloat16),
                preferred_element_type=jnp.float32) + b2_ref[...], 0.0)
    h = jnp.maximum(
        jnp.dot(h.astype(jnp.bfloat16), W3_ref[...].astype(jnp.bfloat16),
                preferred_element_type=jnp.float32) + b3_ref[...], 0.0)
    deep = jnp.dot(h, Wo_ref[...], preferred_element_type=jnp.float32) + bo_ref[0, 0]
    out_ref[...] = jax.nn.sigmoid(0.5 * (fm + deep))


_BLK = 1024
_H1, _H2, _H3 = 1024, 512, 256


def _full(shape):
    n = len(shape)
    return pl.BlockSpec(shape, lambda i, n=n: (0,) * n)


_tc_call = pl.pallas_call(
    _tc_body,
    grid=(B // _BLK,),
    in_specs=[
        pl.BlockSpec((SPARSE_DIM, _BLK, 128), lambda i: (0, i, 0)),
        pl.BlockSpec((_BLK, DENSE_DIM + SPARSE_DIM), lambda i: (i, 0)),
        _full((1, 1)),
        _full((FN, 1)),
        _full((FN, 64)),
        _full((FN, _H1)),
        _full((1, _H1)),
        _full((_H1, _H2)),
        _full((1, _H2)),
        _full((_H2, _H3)),
        _full((1, _H3)),
        _full((_H3, 1)),
        _full((1, 1)),
    ],
    out_specs=pl.BlockSpec((_BLK, 1), lambda i: (i, 0)),
    out_shape=jax.ShapeDtypeStruct((B, 1), jnp.float32),
    compiler_params=pltpu.CompilerParams(
        dimension_semantics=("arbitrary",)),
)


def kernel(inputs, tables, w0, w, V, W1, b1, W2, b2, W3, b3, Wo, bo):
    table_pad = jnp.pad(tables, ((0, 0), (0, 0), (0, 128 - EMB)))
    idx3 = (inputs[:, DENSE_DIM:].astype(jnp.int32).T
            .reshape(SPARSE_DIM, 8, 128))
    emb3 = _sc_gather(table_pad, idx3)
    out = _tc_call(emb3, inputs, w0.reshape(1, 1), w, V, W1,
                   b1.reshape(1, _H1), W2, b2.reshape(1, _H2), W3,
                   b3.reshape(1, _H3), Wo, bo.reshape(1, 1))
    return out
